# baseline (speedup 1.0000x reference)
import jax
import jax.numpy as jnp
from jax.experimental import pallas as pl

_C = 80
_TOPK = 13
_SOFT_RADIUS = 3.0


def _fcos_loss_body(pcls_ref, pbox_ref, gt_ref, gtt_ref, aux_ref, valid_ref,
                    out_ref):
    C, M = pcls_ref.shape[1], pcls_ref.shape[2]
    N = gt_ref.shape[1]
    for img in range(pcls_ref.shape[0]):
        _one_image(pcls_ref[img], pbox_ref[img], gt_ref[img], gtt_ref[img],
                   aux_ref, valid_ref[img], out_ref, img, C, M, N)


def _one_image(p, pb, gt, gtt, aux_ref, validf, out_ref, img, C, M, N):
    ax = aux_ref[0:1, :]
    ay = aux_ref[1:2, :]
    ist = aux_ref[2:3, :]

    gx0 = gt[:, 0:1]
    gy0 = gt[:, 1:2]
    gx1 = gt[:, 2:3]
    gy1 = gt[:, 3:4]
    glab = gt[:, 4:5]
    keep = jnp.minimum(gx1 - gx0, gy1 - gy0) >= 8.0

    px0 = pb[0:1, :]
    py0 = pb[1:2, :]
    px1 = pb[2:3, :]
    py1 = pb[3:4, :]

    gcx = (gx0 + gx1) * 0.5
    gcy = (gy0 + gy1) * 0.5
    dx = ax - gcx
    dy = ay - gcy
    dist = jnp.sqrt(dx * dx + dy * dy) * ist
    prior = jnp.power(10.0, dist - _SOFT_RADIUS)

    tlx = jnp.maximum(px0, gx0)
    tly = jnp.maximum(py0, gy0)
    brx = jnp.minimum(px1, gx1)
    bry = jnp.minimum(py1, gy1)
    iw = jnp.clip(brx - tlx, 0.0, None)
    ih = jnp.clip(bry - tly, 0.0, None)
    inter = iw * ih
    area_a = jnp.clip(px1 - px0, 0.0, None) * jnp.clip(py1 - py0, 0.0, None)
    area_b = (gx1 - gx0) * (gy1 - gy0)
    union = jnp.clip(area_a + area_b - inter, 1e-7, None)
    iou = inter / union

    lane_c = jax.lax.broadcasted_iota(jnp.int32, (N, C), 1)
    onehot = (lane_c == glab.astype(jnp.int32)).astype(jnp.float32)
    logits = jnp.dot(onehot, p, preferred_element_type=jnp.float32,
                     precision=jax.lax.Precision.HIGHEST)
    pred_scores = jax.nn.sigmoid(logits)
    dlt = pred_scores - iou
    scale = dlt * dlt
    lse = jnp.logaddexp(0.0, logits)
    cls_cost = (lse - logits * iou) * scale
    iou_cost = -jnp.log(iou + 1e-7) * 3.0
    inf = jnp.float32(jnp.inf)
    cost = jnp.where(keep, cls_cost + iou_cost + prior, inf)

    iouz = jnp.where(keep, iou, 0.0)

    w = iouz
    acc = jnp.zeros((N, 1), jnp.float32)
    for r in range(_TOPK):
        mx = jnp.max(w, axis=1, keepdims=True)
        acc = acc + jnp.maximum(mx, 0.0)
        if r + 1 < _TOPK:
            w = jnp.where(w == mx, -1.0, w)
    ks = jnp.where(keep, jnp.maximum(acc.astype(jnp.int32), 1), 0)

    w = cost
    mns = []
    for r in range(_TOPK):
        mn = jnp.min(w, axis=1, keepdims=True)
        mns.append(mn)
        if r + 1 < _TOPK:
            w = jnp.where(w == mn, inf, w)
    mnmat = jnp.concatenate(mns, axis=1)
    ki = jax.lax.broadcasted_iota(jnp.int32, (N, _TOPK), 1)
    khot = (ki == (ks - 1)).astype(jnp.float32)
    thr = jnp.sum(jnp.where(khot > 0.5, mnmat, 0.0), axis=1, keepdims=True)
    match = (cost <= thr).astype(jnp.float32)

    ones_row = jnp.ones((1, N), jnp.float32)
    cntf = jnp.dot(ones_row, match, preferred_element_type=jnp.float32,
                   precision=jax.lax.Precision.HIGHEST)
    multi = cntf > 1.0
    gidx = jax.lax.broadcasted_iota(jnp.int32, (N, M), 0)
    mnc = jnp.min(cost, axis=0, keepdims=True)
    firstj = jnp.min(jnp.where(cost == mnc, gidx, N), axis=0, keepdims=True)
    mf = jnp.where(multi, (gidx == firstj).astype(jnp.float32), match)

    sums = jnp.dot(gtt, mf, preferred_element_type=jnp.float32,
                   precision=jax.lax.Precision.HIGHEST)
    fg = sums[0:1, :] > 0.0
    tx0 = sums[1:2, :]
    ty0 = sums[2:3, :]
    tx1 = sums[3:4, :]
    ty1 = sums[4:5, :]
    metrics = jnp.sum(mf * iou, axis=0, keepdims=True)
    x = jnp.sum(mf * logits, axis=0, keepdims=True)
    met_sum = jnp.sum(metrics)

    sig = jax.nn.sigmoid(p)
    ce = jnp.logaddexp(0.0, p) * sig * sig
    base = jnp.sum(ce * validf)
    sigx = jax.nn.sigmoid(x)
    lsex = jnp.logaddexp(0.0, x)
    ce_at = lsex * sigx * sigx
    sf = jnp.abs(metrics - sigx)
    pos_ce = (lsex - x * metrics) * sf * sf
    posf = jnp.where(fg, validf, 0.0)
    cls_num = base + jnp.sum(posf * (pos_ce - ce_at))

    t2lx = jnp.maximum(px0, tx0)
    t2ly = jnp.maximum(py0, ty0)
    b2rx = jnp.minimum(px1, tx1)
    b2ry = jnp.minimum(py1, ty1)
    w2 = jnp.maximum(b2rx - t2lx, 0.0)
    h2 = jnp.maximum(b2ry - t2ly, 0.0)
    inter2 = w2 * h2
    area_p = jnp.maximum(px1 - px0, 0.0) * jnp.maximum(py1 - py0, 0.0)
    area_t = (tx1 - tx0) * (ty1 - ty0)
    union2 = area_p + area_t - inter2
    iou2 = inter2 / jnp.maximum(union2, 1e-7)
    cw = jnp.maximum(jnp.maximum(px1, tx1) - jnp.minimum(px0, tx0), 0.0)
    chh = jnp.maximum(jnp.maximum(py1, ty1) - jnp.minimum(py0, ty0), 0.0)
    carea = cw * chh
    giou = iou2 - (carea - union2) / jnp.maximum(carea, 1e-7)
    reg_num = jnp.sum((1.0 - giou) * metrics)

    out_ref[img] = jnp.concatenate([
        jnp.full((1, 128), cls_num, jnp.float32),
        jnp.full((1, 128), reg_num, jnp.float32),
        jnp.full((1, 128), met_sum, jnp.float32),
        jnp.zeros((5, 128), jnp.float32),
    ], axis=0)


def kernel(pred_cls, pred_box, gt_boxes, gt_labels, anchors, strides, mask):
    B, M, C = pred_cls.shape
    N = gt_labels.shape[1]
    pcls_t = jnp.transpose(pred_cls, (0, 2, 1))
    pbox_t = jnp.transpose(pred_box, (0, 2, 1))
    glabf = gt_labels.astype(jnp.float32)
    gtall = jnp.concatenate(
        [gt_boxes, glabf[..., None], jnp.zeros((B, N, 3), jnp.float32)],
        axis=2)
    gtt = jnp.concatenate(
        [jnp.ones((B, 1, N), jnp.float32),
         jnp.transpose(gt_boxes, (0, 2, 1)),
         jnp.zeros((B, 3, N), jnp.float32)], axis=1)
    aux = jnp.concatenate(
        [anchors.T, 1.0 / strides[None, :], jnp.zeros((5, M), jnp.float32)],
        axis=0)
    validf = (~mask).astype(jnp.float32)[:, None, :]

    out = pl.pallas_call(
        _fcos_loss_body,
        grid=(B // 4,),
        in_specs=[
            pl.BlockSpec((4, C, M), lambda b: (b, 0, 0)),
            pl.BlockSpec((4, 4, M), lambda b: (b, 0, 0)),
            pl.BlockSpec((4, N, 8), lambda b: (b, 0, 0)),
            pl.BlockSpec((4, 8, N), lambda b: (b, 0, 0)),
            pl.BlockSpec((8, M), lambda b: (0, 0)),
            pl.BlockSpec((4, 1, M), lambda b: (b, 0, 0)),
        ],
        out_specs=pl.BlockSpec((4, 8, 128), lambda b: (b, 0, 0)),
        out_shape=jax.ShapeDtypeStruct((B, 8, 128), jnp.float32),
    )(pcls_t, pbox_t, gtall, gtt, aux, validf)

    cls_total = jnp.sum(out[:, 0, 0])
    reg_total = jnp.sum(out[:, 1, 0])
    met_total = jnp.sum(out[:, 2, 0])
    num_fgs = jnp.maximum(met_total, 1.0)
    loss_cls = cls_total / num_fgs
    loss_reg = reg_total / num_fgs
    total = loss_cls * 1.0 + loss_reg * 2.0
    return (loss_cls, loss_reg, total)


# speedup vs baseline: 1.0123x; 1.0123x over previous
import jax
import jax.numpy as jnp
from jax.experimental import pallas as pl

_C = 80
_TOPK = 13
_SOFT_RADIUS = 3.0


def _fcos_loss_body(pcls_ref, pbox_ref, gt_ref, gtt_ref, aux_ref, valid_ref,
                    out_ref):
    C, M = pcls_ref.shape[1], pcls_ref.shape[2]
    N = gt_ref.shape[1]
    for img in range(pcls_ref.shape[0]):
        _one_image(pcls_ref[img], pbox_ref[img], gt_ref[img], gtt_ref[img],
                   aux_ref, valid_ref[img], out_ref, img, C, M, N)


def _one_image(p, pb, gt, gtt, aux_ref, validf, out_ref, img, C, M, N):
    ax = aux_ref[0:1, :]
    ay = aux_ref[1:2, :]
    ist = aux_ref[2:3, :]

    gx0 = gt[:, 0:1]
    gy0 = gt[:, 1:2]
    gx1 = gt[:, 2:3]
    gy1 = gt[:, 3:4]
    glab = gt[:, 4:5]
    keep = jnp.minimum(gx1 - gx0, gy1 - gy0) >= 8.0

    px0 = pb[0:1, :]
    py0 = pb[1:2, :]
    px1 = pb[2:3, :]
    py1 = pb[3:4, :]

    gcx = (gx0 + gx1) * 0.5
    gcy = (gy0 + gy1) * 0.5
    dx = ax - gcx
    dy = ay - gcy
    dist = jnp.sqrt(dx * dx + dy * dy) * ist
    prior = jnp.power(10.0, dist - _SOFT_RADIUS)

    tlx = jnp.maximum(px0, gx0)
    tly = jnp.maximum(py0, gy0)
    brx = jnp.minimum(px1, gx1)
    bry = jnp.minimum(py1, gy1)
    iw = jnp.clip(brx - tlx, 0.0, None)
    ih = jnp.clip(bry - tly, 0.0, None)
    inter = iw * ih
    area_a = jnp.clip(px1 - px0, 0.0, None) * jnp.clip(py1 - py0, 0.0, None)
    area_b = (gx1 - gx0) * (gy1 - gy0)
    union = jnp.clip(area_a + area_b - inter, 1e-7, None)
    iou = inter / union

    lane_c = jax.lax.broadcasted_iota(jnp.int32, (N, C), 1)
    onehot = (lane_c == glab.astype(jnp.int32)).astype(jnp.float32)
    logits = jnp.dot(onehot, p, preferred_element_type=jnp.float32,
                     precision=jax.lax.Precision.HIGHEST)
    pred_scores = jax.nn.sigmoid(logits)
    dlt = pred_scores - iou
    scale = dlt * dlt
    lse = jnp.logaddexp(0.0, logits)
    cls_cost = (lse - logits * iou) * scale
    iou_cost = -jnp.log(iou + 1e-7) * 3.0
    inf = jnp.float32(jnp.inf)
    cost = jnp.where(keep, cls_cost + iou_cost + prior, inf)

    iouz = jnp.where(keep, iou, 0.0)

    w = iouz
    acc = jnp.zeros((N, 1), jnp.float32)
    for r in range(_TOPK):
        mx = jnp.max(w, axis=1, keepdims=True)
        acc = acc + jnp.maximum(mx, 0.0)
        if r + 1 < _TOPK:
            w = jnp.where(w == mx, -1.0, w)
    ks = jnp.where(keep, jnp.maximum(acc.astype(jnp.int32), 1), 0)

    w = cost
    mns = []
    for r in range(_TOPK):
        mn = jnp.min(w, axis=1, keepdims=True)
        mns.append(mn)
        if r + 1 < _TOPK:
            w = jnp.where(w == mn, inf, w)
    mnmat = jnp.concatenate(mns, axis=1)
    ki = jax.lax.broadcasted_iota(jnp.int32, (N, _TOPK), 1)
    khot = (ki == (ks - 1)).astype(jnp.float32)
    thr = jnp.sum(jnp.where(khot > 0.5, mnmat, 0.0), axis=1, keepdims=True)
    match = (cost <= thr).astype(jnp.float32)

    ones_row = jnp.ones((1, N), jnp.float32)
    cntf = jnp.dot(ones_row, match, preferred_element_type=jnp.float32,
                   precision=jax.lax.Precision.HIGHEST)
    multi = cntf > 1.0
    gidx = jax.lax.broadcasted_iota(jnp.int32, (N, M), 0)
    mnc = jnp.min(cost, axis=0, keepdims=True)
    firstj = jnp.min(jnp.where(cost == mnc, gidx, N), axis=0, keepdims=True)
    mf = jnp.where(multi, (gidx == firstj).astype(jnp.float32), match)

    sums = jnp.dot(gtt, mf, preferred_element_type=jnp.float32,
                   precision=jax.lax.Precision.HIGHEST)
    fg = sums[0:1, :] > 0.0
    tx0 = sums[1:2, :]
    ty0 = sums[2:3, :]
    tx1 = sums[3:4, :]
    ty1 = sums[4:5, :]
    metrics = jnp.sum(mf * iou, axis=0, keepdims=True)
    x = jnp.sum(mf * logits, axis=0, keepdims=True)
    met_sum = jnp.sum(metrics)

    sig = jax.nn.sigmoid(p)
    ce = jnp.logaddexp(0.0, p) * sig * sig
    base = jnp.sum(ce * validf)
    sigx = jax.nn.sigmoid(x)
    lsex = jnp.logaddexp(0.0, x)
    ce_at = lsex * sigx * sigx
    sf = jnp.abs(metrics - sigx)
    pos_ce = (lsex - x * metrics) * sf * sf
    posf = jnp.where(fg, validf, 0.0)
    cls_num = base + jnp.sum(posf * (pos_ce - ce_at))

    t2lx = jnp.maximum(px0, tx0)
    t2ly = jnp.maximum(py0, ty0)
    b2rx = jnp.minimum(px1, tx1)
    b2ry = jnp.minimum(py1, ty1)
    w2 = jnp.maximum(b2rx - t2lx, 0.0)
    h2 = jnp.maximum(b2ry - t2ly, 0.0)
    inter2 = w2 * h2
    area_p = jnp.maximum(px1 - px0, 0.0) * jnp.maximum(py1 - py0, 0.0)
    area_t = (tx1 - tx0) * (ty1 - ty0)
    union2 = area_p + area_t - inter2
    iou2 = inter2 / jnp.maximum(union2, 1e-7)
    cw = jnp.maximum(jnp.maximum(px1, tx1) - jnp.minimum(px0, tx0), 0.0)
    chh = jnp.maximum(jnp.maximum(py1, ty1) - jnp.minimum(py0, ty0), 0.0)
    carea = cw * chh
    giou = iou2 - (carea - union2) / jnp.maximum(carea, 1e-7)
    reg_num = jnp.sum((1.0 - giou) * metrics)

    out_ref[img] = jnp.concatenate([
        jnp.full((1, 128), cls_num, jnp.float32),
        jnp.full((1, 128), reg_num, jnp.float32),
        jnp.full((1, 128), met_sum, jnp.float32),
        jnp.zeros((5, 128), jnp.float32),
    ], axis=0)


def kernel(pred_cls, pred_box, gt_boxes, gt_labels, anchors, strides, mask):
    B, M, C = pred_cls.shape
    N = gt_labels.shape[1]
    pcls_t = jnp.transpose(pred_cls, (0, 2, 1))
    pbox_t = jnp.transpose(pred_box, (0, 2, 1))
    glabf = gt_labels.astype(jnp.float32)
    gtall = jnp.concatenate(
        [gt_boxes, glabf[..., None], jnp.zeros((B, N, 3), jnp.float32)],
        axis=2)
    gtt = jnp.concatenate(
        [jnp.ones((B, 1, N), jnp.float32),
         jnp.transpose(gt_boxes, (0, 2, 1)),
         jnp.zeros((B, 3, N), jnp.float32)], axis=1)
    aux = jnp.concatenate(
        [anchors.T, 1.0 / strides[None, :], jnp.zeros((5, M), jnp.float32)],
        axis=0)
    validf = (~mask).astype(jnp.float32)[:, None, :]

    out = pl.pallas_call(
        _fcos_loss_body,
        grid=(B // 2,),
        in_specs=[
            pl.BlockSpec((2, C, M), lambda b: (b, 0, 0)),
            pl.BlockSpec((2, 4, M), lambda b: (b, 0, 0)),
            pl.BlockSpec((2, N, 8), lambda b: (b, 0, 0)),
            pl.BlockSpec((2, 8, N), lambda b: (b, 0, 0)),
            pl.BlockSpec((8, M), lambda b: (0, 0)),
            pl.BlockSpec((2, 1, M), lambda b: (b, 0, 0)),
        ],
        out_specs=pl.BlockSpec((2, 8, 128), lambda b: (b, 0, 0)),
        out_shape=jax.ShapeDtypeStruct((B, 8, 128), jnp.float32),
    )(pcls_t, pbox_t, gtall, gtt, aux, validf)

    cls_total = jnp.sum(out[:, 0, 0])
    reg_total = jnp.sum(out[:, 1, 0])
    met_total = jnp.sum(out[:, 2, 0])
    num_fgs = jnp.maximum(met_total, 1.0)
    loss_cls = cls_total / num_fgs
    loss_reg = reg_total / num_fgs
    total = loss_cls * 1.0 + loss_reg * 2.0
    return (loss_cls, loss_reg, total)
